# baseline (speedup 1.0000x reference)
import jax
import jax.numpy as jnp
import numpy as np
from jax.experimental import pallas as pl
from jax.experimental.pallas import tpu as pltpu

_N = 100
_K = 50
_SWEEPS = 7
_RADIUS = 1.2
_BIG = 1e6


_ROUNDS = _SWEEPS * (_N - 1)


def _isomap_mlp_body(x_ref, xT_ref, W1_ref, b1_ref, W2_ref, b2_ref, out_ref):
    f32 = jnp.float32
    row_i = jax.lax.broadcasted_iota(jnp.int32, (_N, _N), 0)
    col_i = jax.lax.broadcasted_iota(jnp.int32, (_N, _N), 1)
    eye_b = row_i == col_i
    eyef = jnp.where(eye_b, 1.0, 0.0).astype(f32)
    offmask = jnp.where((col_i - row_i == _K) | (row_i - col_i == _K), 1.0, 0.0).astype(f32)
    sgn_col = jnp.where(jax.lax.broadcasted_iota(jnp.int32, (_N, 1), 0) < _K, 1.0, -1.0).astype(f32)
    sgn_row = jnp.where(jax.lax.broadcasted_iota(jnp.int32, (1, _N), 1) < _K, 1.0, -1.0).astype(f32)

    xi0 = x_ref[:, 0:1]
    xi1 = x_ref[:, 1:2]
    xj0 = xT_ref[0:1, :]
    xj1 = xT_ref[1:2, :]
    d2 = (xi0 - xj0) ** 2 + (xi1 - xj1) ** 2
    D = jnp.sqrt(jnp.maximum(d2, 1e-12))
    D = jnp.where(eye_b, 0.0, D)
    G = jnp.where(D <= _RADIUS, D, _BIG)
    G = jnp.where(eye_b, 0.0, G)

    def fw_step(k, G):
        col = jnp.sum(jnp.where(col_i == k, G, 0.0), axis=1, keepdims=True)
        row = jnp.sum(jnp.where(row_i == k, G, 0.0), axis=0, keepdims=True)
        return jnp.minimum(G, col + row)

    G = jax.lax.fori_loop(0, _N, fw_step, G)
    finite = G < (_BIG * 0.5)
    fmax = jnp.max(jnp.where(finite, G, 0.0))
    G = jnp.where(finite, G, fmax)

    G2 = G * G
    rm = jnp.mean(G2, axis=1, keepdims=True)
    cm = jnp.mean(G2, axis=0, keepdims=True)
    tm = jnp.mean(G2)
    B = -0.5 * (G2 - rm - cm + tm)

    def perm_c(M):
        return jnp.concatenate(
            [M[:, 0:1], M[:, _K:_K + 1], M[:, 1:_K - 1], M[:, _K + 1:_N],
             M[:, _K - 1:_K]], axis=1)

    def coeffs(d, o, dsw, sgnhalf):
        tau = (dsw - d) / (2.0 * o) * sgnhalf
        sg = jnp.where(tau >= 0, 1.0, -1.0)
        t = sg / (jnp.abs(tau) + jnp.sqrt(1.0 + tau * tau))
        t = jnp.where(jnp.abs(o) <= 1e-30, 0.0, t)
        c = jax.lax.rsqrt(1.0 + t * t)
        s = t * c
        return c, s

    eyept = jnp.where(
        ((col_i == 0) & (row_i == 0))
        | ((col_i == 1) & (row_i == _K))
        | ((col_i >= 2) & (col_i <= _K - 1) & (row_i == col_i - 1))
        | ((col_i >= _K) & (col_i <= _N - 2) & (row_i == col_i + 1))
        | ((col_i == _N - 1) & (row_i == _K - 1)),
        1.0, 0.0).astype(f32)
    offpt = jnp.where(
        ((col_i == 0) & (row_i == _K))
        | ((col_i == 1) & (row_i == 0))
        | ((col_i >= 2) & (col_i <= _K - 1) & (row_i == col_i + _K - 1))
        | ((col_i >= _K) & (col_i <= _N - 2) & (row_i == col_i - _K + 1))
        | ((col_i == _N - 1) & (row_i == _N - 1)),
        1.0, 0.0).astype(f32)

    def round_body(r, carry):
        A, V = carry
        d_row = jnp.sum(A * eyef, axis=0, keepdims=True)
        o_row = jnp.sum(A * offmask, axis=0, keepdims=True)
        c_row, s_row = coeffs(d_row, o_row,
                              jnp.concatenate([d_row[:, _K:], d_row[:, :_K]], axis=1),
                              sgn_row)
        coef2_row = -sgn_row * s_row
        cpr = perm_c(c_row)
        c2pr = perm_c(coef2_row)
        Mt = eyept * cpr + offpt * c2pr
        T1 = jax.lax.dot_general(Mt, A, (((0,), (0,)), ((), ())),
                                 preferred_element_type=f32)
        A = jnp.dot(T1, Mt, preferred_element_type=f32)
        V = jnp.dot(V, Mt, preferred_element_type=f32)
        return A, V

    A, V = jax.lax.fori_loop(0, _ROUNDS, round_body, (B, eyef))

    w = jnp.sum(A * eyef, axis=1, keepdims=True)
    m1 = jnp.max(w)
    is1 = w == m1
    w_rest = jnp.where(is1, -1e30, w)
    m2 = jnp.max(w_rest)
    is2 = w_rest == m2
    s1 = jnp.sqrt(jnp.maximum(m1, 1e-12))
    s2 = jnp.sqrt(jnp.maximum(m2, 1e-12))
    ST = jnp.concatenate(
        [jnp.where(is1, s1, 0.0), jnp.where(is2, s2, 0.0)], axis=1)
    emb = jnp.dot(V, ST, preferred_element_type=f32)

    h = jnp.maximum(
        jnp.dot(emb, W1_ref[...], preferred_element_type=f32) + b1_ref[...],
        0.0)
    out_ref[...] = jnp.dot(h, W2_ref[...], preferred_element_type=f32) + b2_ref[...]


def kernel(x, W1, b1, W2, b2):
    x = x.reshape(_N, 2).astype(jnp.float32)
    xT = x.T
    return pl.pallas_call(
        _isomap_mlp_body,
        out_shape=jax.ShapeDtypeStruct((_N, 10), jnp.float32),
    )(x, xT, W1, b1.reshape(1, 512), W2, b2.reshape(1, 10))


# speedup vs baseline: 1.5831x; 1.5831x over previous
import jax
import jax.numpy as jnp
import numpy as np
from jax.experimental import pallas as pl
from jax.experimental.pallas import tpu as pltpu

_N = 100
_K = 50
_SWEEPS = 6
_RADIUS = 1.2
_BIG = 1e6


_ROUNDS = _SWEEPS * (_N - 1)


def _isomap_mlp_body(x_ref, xT_ref, W1_ref, b1_ref, W2_ref, b2_ref, out_ref):
    f32 = jnp.float32
    row_i = jax.lax.broadcasted_iota(jnp.int32, (_N, _N), 0)
    col_i = jax.lax.broadcasted_iota(jnp.int32, (_N, _N), 1)
    eye_b = row_i == col_i
    eyef = jnp.where(eye_b, 1.0, 0.0).astype(f32)
    offmask = jnp.where((col_i - row_i == _K) | (row_i - col_i == _K), 1.0, 0.0).astype(f32)
    sgn_col = jnp.where(jax.lax.broadcasted_iota(jnp.int32, (_N, 1), 0) < _K, 1.0, -1.0).astype(f32)
    sgn_row = jnp.where(jax.lax.broadcasted_iota(jnp.int32, (1, _N), 1) < _K, 1.0, -1.0).astype(f32)

    xi0 = x_ref[:, 0:1]
    xi1 = x_ref[:, 1:2]
    xj0 = xT_ref[0:1, :]
    xj1 = xT_ref[1:2, :]
    d2 = (xi0 - xj0) ** 2 + (xi1 - xj1) ** 2
    D = jnp.sqrt(jnp.maximum(d2, 1e-12))
    D = jnp.where(eye_b, 0.0, D)
    G = jnp.where(D <= _RADIUS, D, _BIG)
    G = jnp.where(eye_b, 0.0, G)

    def fw_step(k, G):
        col = jnp.sum(jnp.where(col_i == k, G, 0.0), axis=1, keepdims=True)
        row = jnp.sum(jnp.where(row_i == k, G, 0.0), axis=0, keepdims=True)
        return jnp.minimum(G, col + row)

    G = jax.lax.fori_loop(0, _N, fw_step, G)
    finite = G < (_BIG * 0.5)
    fmax = jnp.max(jnp.where(finite, G, 0.0))
    G = jnp.where(finite, G, fmax)

    G2 = G * G
    rm = jnp.mean(G2, axis=1, keepdims=True)
    cm = jnp.mean(G2, axis=0, keepdims=True)
    tm = jnp.mean(G2)
    B = -0.5 * (G2 - rm - cm + tm)

    def perm_c(M):
        return jnp.concatenate(
            [M[:, 0:1], M[:, _K:_K + 1], M[:, 1:_K - 1], M[:, _K + 1:_N],
             M[:, _K - 1:_K]], axis=1)

    def coeffs(d, o, dsw, sgnhalf):
        tau = (dsw - d) / (2.0 * o) * sgnhalf
        sg = jnp.where(tau >= 0, 1.0, -1.0)
        t = sg / (jnp.abs(tau) + jnp.sqrt(1.0 + tau * tau))
        t = jnp.where(jnp.abs(o) <= 1e-30, 0.0, t)
        c = jax.lax.rsqrt(1.0 + t * t)
        s = t * c
        return c, s

    def perm_r(M):
        return jnp.concatenate(
            [M[0:1, :], M[_K:_K + 1, :], M[1:_K - 1, :], M[_K + 1:_N, :],
             M[_K - 1:_K, :]], axis=0)

    def perm_hr(M):
        return jnp.concatenate(
            [M[_K:_K + 1, :], M[0:1, :], M[_K + 1:_N - 1, :], M[1:_K, :],
             M[_N - 1:_N, :]], axis=0)

    def perm_hc(M):
        return jnp.concatenate(
            [M[:, _K:_K + 1], M[:, 0:1], M[:, _K + 1:_N - 1], M[:, 1:_K],
             M[:, _N - 1:_N]], axis=1)

    def round_body(r, carry):
        A, V = carry
        d_row = jnp.sum(A * eyef, axis=0, keepdims=True)
        o_row = jnp.sum(A * offmask, axis=0, keepdims=True)
        c_row, s_row = coeffs(d_row, o_row,
                              jnp.concatenate([d_row[:, _K:], d_row[:, :_K]], axis=1),
                              sgn_row)
        coef2_row = -sgn_row * s_row
        cpr = perm_c(c_row)
        c2pr = perm_c(coef2_row)
        cpc = jnp.transpose(cpr)
        c2pc = jnp.transpose(c2pr)
        R1 = perm_r(A)
        R2 = perm_hr(A)
        A = (cpc * (perm_c(R1) * cpr + perm_hc(R1) * c2pr)
             + c2pc * (perm_c(R2) * cpr + perm_hc(R2) * c2pr))
        V = perm_c(V) * cpr + perm_hc(V) * c2pr
        return A, V

    def triple_round(r, carry):
        return round_body(r, round_body(r, round_body(r, carry)))

    A, V = jax.lax.fori_loop(0, _ROUNDS // 3, triple_round, (B, eyef))

    w = jnp.sum(A * eyef, axis=1, keepdims=True)
    m1 = jnp.max(w)
    is1 = w == m1
    w_rest = jnp.where(is1, -1e30, w)
    m2 = jnp.max(w_rest)
    is2 = w_rest == m2
    s1 = jnp.sqrt(jnp.maximum(m1, 1e-12))
    s2 = jnp.sqrt(jnp.maximum(m2, 1e-12))
    ST = jnp.concatenate(
        [jnp.where(is1, s1, 0.0), jnp.where(is2, s2, 0.0)], axis=1)
    emb = jnp.dot(V, ST, preferred_element_type=f32)

    h = jnp.maximum(
        jnp.dot(emb, W1_ref[...], preferred_element_type=f32) + b1_ref[...],
        0.0)
    out_ref[...] = jnp.dot(h, W2_ref[...], preferred_element_type=f32) + b2_ref[...]


def kernel(x, W1, b1, W2, b2):
    x = x.reshape(_N, 2).astype(jnp.float32)
    xT = x.T
    return pl.pallas_call(
        _isomap_mlp_body,
        out_shape=jax.ShapeDtypeStruct((_N, 10), jnp.float32),
    )(x, xT, W1, b1.reshape(1, 512), W2, b2.reshape(1, 10))
